# baseline (speedup 1.0000x reference)
import jax
import jax.numpy as jnp
from jax import lax
from jax.experimental import pallas as pl
from jax.experimental.pallas import tpu as pltpu
from jax.experimental.pallas import tpu_sc as plsc

N = 10000
F = 128
E = 320000
CH = 128
NCHUNKS = E // CH
NC, NS, L = 2, 16, 16
NW = NC * NS
RPT = 624
TAIL = N - RPT * NS
MAIN = NCHUNKS // NW
XTRA = NCHUNKS - MAIN * NW


def _sc_aggregate(with_cnt: bool):
    mesh = plsc.VectorSubcoreMesh(
        core_axis_name="c", subcore_axis_name="s", num_cores=NC, num_subcores=NS
    )
    out_type = [jax.ShapeDtypeStruct((NC * N, F), jnp.float32)]
    scratch = [
        pltpu.VMEM((4, CH), jnp.int32),
        pltpu.VMEM((4, CH), jnp.int32),
        pltpu.VMEM((CH, F), jnp.float32),
        pltpu.VMEM((CH, F), jnp.float32),
        pltpu.VMEM_SHARED((N, F), jnp.float32),
        pltpu.SemaphoreType.DMA,
        pltpu.SemaphoreType.DMA,
        pltpu.SemaphoreType.DMA,
        pltpu.SemaphoreType.DMA,
        pltpu.SemaphoreType.DMA,
        pltpu.SemaphoreType.DMA,
        pltpu.SemaphoreType.DMA,
        pltpu.SemaphoreType.DMA,
    ]
    if with_cnt:
        out_type.append(jax.ShapeDtypeStruct((NW * N,), jnp.float32))
        scratch.append(pltpu.VMEM((N,), jnp.float32))

    def body(x_hbm, es_hbm, ed_hbm, sums_out, *rest):
        if with_cnt:
            (cnt_out, es_r, ed_r, r0, r1, shared,
             i0, i1, i2, i3, g0, g1, s0, s1, cnt_v) = rest
        else:
            (es_r, ed_r, r0, r1, shared, i0, i1, i2, i3, g0, g1, s0, s1) = rest
            cnt_out = cnt_v = None
        rows = (r0, r1)
        isem = (i0, i1, i2, i3)
        gsem = (g0, g1)
        ssem = (s0, s1)

        cid = lax.axis_index("c")
        sid = lax.axis_index("s")
        wid = sid * NC + cid

        zeros16 = jnp.zeros((L,), jnp.float32)

        def zrow(r, _):
            for j in range(F // L):
                r0[r, pl.ds(j * L, L)] = zeros16
            return 0
        lax.fori_loop(0, CH, zrow, 0)

        def zero_span(lo, ln):
            off = 0
            while ln > 0:
                step = min(ln, CH)
                pltpu.sync_copy(r0.at[pl.ds(0, step)],
                                shared.at[pl.ds(lo + off, step)])
                off += step
                ln -= step

        zero_span(sid * RPT, RPT)

        @pl.when(sid == NS - 1)
        def _():
            zero_span(NS * RPT, TAIL)

        if with_cnt:
            def zcnt(k, _):
                cnt_v[pl.ds(k * L, L)] = zeros16
                return 0
            lax.fori_loop(0, N // L, zcnt, 0)
        plsc.subcore_barrier()

        ones16 = jnp.ones((L,), jnp.float32)

        def chunk_id(i):
            return wid + i * NW

        def fire_idx(i, q):
            c = chunk_id(i)
            pltpu.async_copy(es_hbm.at[c], es_r.at[q], isem[q])
            pltpu.async_copy(ed_hbm.at[c], ed_r.at[q], isem[q])

        def wait_idx(i, q):
            c = chunk_id(i)
            pltpu.make_async_copy(es_hbm.at[c], es_r.at[q], isem[q]).wait()
            pltpu.make_async_copy(ed_hbm.at[c], ed_r.at[q], isem[q]).wait()

        def fire_gather(q, b):
            pltpu.async_copy(x_hbm.at[es_r.at[q]], rows[b], gsem[b])

        def wait_gather(q, b):
            pltpu.make_async_copy(x_hbm.at[es_r.at[q]], rows[b],
                                  gsem[b]).wait()

        def fire_scatter(q, b):
            pltpu.async_copy(rows[b], shared.at[ed_r.at[q]], ssem[b],
                             add=True)

        def wait_scatter(q, b):
            pltpu.make_async_copy(rows[b], shared.at[ed_r.at[q]],
                                  ssem[b]).wait()

        def count_chunk(q):
            if with_cnt:
                for j in range(CH // L):
                    idx16 = ed_r[q, pl.ds(j * L, L)]
                    plsc.addupdate_scatter(cnt_v, [idx16], ones16)

        def step(i, k, ws, g1, i2):
            b, bn, q = k % 2, (k + 1) % 2, k % 4
            if ws:
                wait_scatter((k - 1) % 4, bn)
            if g1:
                wait_idx(i + 1, (k + 1) % 4)
                fire_gather((k + 1) % 4, bn)
            if i2:
                fire_idx(i + 2, (k + 2) % 4)
            wait_gather(q, b)
            fire_scatter(q, b)
            count_chunk(q)

        fire_idx(0, 0)
        fire_idx(1, 1)
        wait_idx(0, 0)
        fire_gather(0, 0)

        step(0, 0, ws=False, g1=True, i2=True)
        step(1, 1, ws=True, g1=True, i2=True)
        step(2, 2, ws=True, g1=True, i2=True)
        step(3, 3, ws=True, g1=True, i2=True)

        def group(j, _):
            for k in range(4):
                step(4 + 4 * j + k, k, ws=True, g1=True, i2=True)
            return 0
        lax.fori_loop(0, (MAIN - 6) // 4, group, 0)

        step(MAIN - 2, (MAIN - 2) % 4, ws=True, g1=True, i2=False)
        step(MAIN - 1, (MAIN - 1) % 4, ws=True, g1=False, i2=False)
        wait_scatter((MAIN - 1) % 4, (MAIN - 1) % 2)

        @pl.when(wid < XTRA)
        def _():
            c = MAIN * NW + wid
            pltpu.sync_copy(es_hbm.at[c], es_r.at[0])
            pltpu.sync_copy(ed_hbm.at[c], ed_r.at[0])
            pltpu.async_copy(x_hbm.at[es_r.at[0]], rows[0], gsem[0]).wait()
            pltpu.sync_copy(rows[0], shared.at[ed_r.at[0]], add=True)
            count_chunk(0)

        plsc.subcore_barrier()

        if with_cnt:
            pltpu.sync_copy(cnt_v, cnt_out.at[pl.ds(wid * N, N)])
        base = sid * RPT
        pltpu.sync_copy(
            shared.at[pl.ds(base, RPT)],
            sums_out.at[pl.ds(cid * N + base, RPT)],
        )

        @pl.when(sid == NS - 1)
        def _():
            pltpu.sync_copy(
                shared.at[pl.ds(NS * RPT, TAIL)],
                sums_out.at[pl.ds(cid * N + NS * RPT, TAIL)],
            )

    return pl.kernel(
        body, out_type=out_type, mesh=mesh, scratch_types=scratch,
        compiler_params=pltpu.CompilerParams(needs_layout_passes=False),
    )


def _dot(a, b):
    return lax.dot_general(
        a, b, (((1,), (0,)), ((), ())),
        precision=lax.Precision.HIGHEST,
        preferred_element_type=jnp.float32,
    )


def _mean(s_ref, cnt_ref):
    s = s_ref[...]
    s = s[:N] + s[N:]
    deg = jnp.sum(cnt_ref[...], axis=0)
    recip = 1.0 / jnp.maximum(deg, 1.0)
    return s * recip[:, None]


def _tc_layer1(s_ref, cnt_ref, x_ref, wl_ref, wr_ref, b_ref, o_ref):
    mean = _mean(s_ref, cnt_ref)
    h = _dot(mean, wl_ref[...]) + _dot(x_ref[...], wr_ref[...]) + b_ref[...]
    o_ref[...] = jnp.maximum(h, 0.0)


def _tc_layer2(s_ref, cnt_ref, h_ref, wl_ref, wr_ref, b_ref,
               wo_ref, bo_ref, o_ref):
    mean = _mean(s_ref, cnt_ref)
    h2 = _dot(mean, wl_ref[...]) + _dot(h_ref[...], wr_ref[...]) + b_ref[...]
    h2 = jnp.maximum(h2, 0.0)
    o_ref[...] = _dot(h2, wo_ref[...]) + bo_ref[...]


_layer1_call = pl.pallas_call(
    _tc_layer1,
    out_shape=jax.ShapeDtypeStruct((N, F), jnp.float32),
)

_layer2_call = pl.pallas_call(
    _tc_layer2,
    out_shape=jax.ShapeDtypeStruct((N, 64), jnp.float32),
)

_sc_agg_cnt = _sc_aggregate(with_cnt=True)
_sc_agg = _sc_aggregate(with_cnt=False)


def kernel(x, edge_index, W1l, W1r, b1, W2l, W2r, b2, Wo, bo):
    es = edge_index[0].reshape(NCHUNKS, CH)
    ed = edge_index[1].reshape(NCHUNKS, CH)
    sums1, cnt = _sc_agg_cnt(x, es, ed)
    cnt = cnt.reshape(NW, N)
    h1 = _layer1_call(sums1, cnt, x, W1l, W1r, b1.reshape(1, F))
    (sums2,) = _sc_agg(h1, es, ed)
    out = _layer2_call(sums2, cnt, h1, W2l, W2r, b2.reshape(1, F),
                       Wo, bo.reshape(1, 64))
    return out


# speedup vs baseline: 1.0272x; 1.0272x over previous
import jax
import jax.numpy as jnp
from jax import lax
from jax.experimental import pallas as pl
from jax.experimental.pallas import tpu as pltpu
from jax.experimental.pallas import tpu_sc as plsc

N = 10000
F = 128
E = 320000
CH = 128
NCHUNKS = E // CH
NC, NS, L = 2, 16, 16
NW = NC * NS
RPT = 624
TAIL = N - RPT * NS
MAIN = NCHUNKS // NW
XTRA = NCHUNKS - MAIN * NW


def _sc_aggregate(with_cnt: bool):
    mesh = plsc.VectorSubcoreMesh(
        core_axis_name="c", subcore_axis_name="s", num_cores=NC, num_subcores=NS
    )
    out_type = [jax.ShapeDtypeStruct((NC * N, F), jnp.float32)]
    scratch = [
        pltpu.VMEM((4, CH), jnp.int32),
        pltpu.VMEM((4, CH), jnp.int32),
        pltpu.VMEM((CH, F), jnp.float32),
        pltpu.VMEM((CH, F), jnp.float32),
        pltpu.VMEM_SHARED((N, F), jnp.float32),
        pltpu.SemaphoreType.DMA,
        pltpu.SemaphoreType.DMA,
        pltpu.SemaphoreType.DMA,
        pltpu.SemaphoreType.DMA,
        pltpu.SemaphoreType.DMA,
        pltpu.SemaphoreType.DMA,
        pltpu.SemaphoreType.DMA,
        pltpu.SemaphoreType.DMA,
    ]
    if with_cnt:
        out_type.append(jax.ShapeDtypeStruct((NW * N,), jnp.float32))
        scratch.append(pltpu.VMEM((N,), jnp.float32))

    def body(x_hbm, es_hbm, ed_hbm, sums_out, *rest):
        if with_cnt:
            (cnt_out, es_r, ed_r, r0, r1, shared,
             i0, i1, i2, i3, g0, g1, s0, s1, cnt_v) = rest
        else:
            (es_r, ed_r, r0, r1, shared, i0, i1, i2, i3, g0, g1, s0, s1) = rest
            cnt_out = cnt_v = None
        rows = (r0, r1)
        isem = (i0, i1, i2, i3)
        gsem = (g0, g1)
        ssem = (s0, s1)

        cid = lax.axis_index("c")
        sid = lax.axis_index("s")
        wid = sid * NC + cid

        def chunk_id(i):
            return wid + i * NW

        def fire_idx(i, q):
            c = chunk_id(i)
            pltpu.async_copy(es_hbm.at[c], es_r.at[q], isem[q])
            pltpu.async_copy(ed_hbm.at[c], ed_r.at[q], isem[q])

        fire_idx(0, 0)
        fire_idx(1, 1)

        zeros16 = jnp.zeros((L,), jnp.float32)

        def zrow(r, _):
            for j in range(F // L):
                r0[r, pl.ds(j * L, L)] = zeros16
            return 0
        lax.fori_loop(0, CH, zrow, 0)

        def zero_span(lo, ln):
            off = 0
            while ln > 0:
                step = min(ln, CH)
                pltpu.sync_copy(r0.at[pl.ds(0, step)],
                                shared.at[pl.ds(lo + off, step)])
                off += step
                ln -= step

        zero_span(sid * RPT, RPT)

        @pl.when(sid == NS - 1)
        def _():
            zero_span(NS * RPT, TAIL)

        if with_cnt:
            def zcnt(k, _):
                cnt_v[pl.ds(k * L, L)] = zeros16
                return 0
            lax.fori_loop(0, N // L, zcnt, 0)

        ones16 = jnp.ones((L,), jnp.float32)

        def wait_idx(i, q):
            c = chunk_id(i)
            pltpu.make_async_copy(es_hbm.at[c], es_r.at[q], isem[q]).wait()
            pltpu.make_async_copy(ed_hbm.at[c], ed_r.at[q], isem[q]).wait()

        def fire_gather(q, b):
            pltpu.async_copy(x_hbm.at[es_r.at[q]], rows[b], gsem[b])

        def wait_gather(q, b):
            pltpu.make_async_copy(x_hbm.at[es_r.at[q]], rows[b],
                                  gsem[b]).wait()

        def fire_scatter(q, b):
            pltpu.async_copy(rows[b], shared.at[ed_r.at[q]], ssem[b],
                             add=True)

        def wait_scatter(q, b):
            pltpu.make_async_copy(rows[b], shared.at[ed_r.at[q]],
                                  ssem[b]).wait()

        def count_chunk(q):
            if with_cnt:
                for j in range(CH // L):
                    idx16 = ed_r[q, pl.ds(j * L, L)]
                    plsc.addupdate_scatter(cnt_v, [idx16], ones16)

        def step(i, k, ws, g1, i2):
            b, bn, q = k % 2, (k + 1) % 2, k % 4
            if ws:
                wait_scatter((k - 1) % 4, bn)
            if g1:
                wait_idx(i + 1, (k + 1) % 4)
                fire_gather((k + 1) % 4, bn)
            if i2:
                fire_idx(i + 2, (k + 2) % 4)
            wait_gather(q, b)
            fire_scatter(q, b)
            count_chunk(q)

        wait_idx(0, 0)
        fire_gather(0, 0)
        plsc.subcore_barrier()

        step(0, 0, ws=False, g1=True, i2=True)
        step(1, 1, ws=True, g1=True, i2=True)
        step(2, 2, ws=True, g1=True, i2=True)
        step(3, 3, ws=True, g1=True, i2=True)

        def group(j, _):
            for k in range(4):
                step(4 + 4 * j + k, k, ws=True, g1=True, i2=True)
            return 0
        lax.fori_loop(0, (MAIN - 6) // 4, group, 0)

        step(MAIN - 2, (MAIN - 2) % 4, ws=True, g1=True, i2=False)
        step(MAIN - 1, (MAIN - 1) % 4, ws=True, g1=False, i2=False)
        wait_scatter((MAIN - 1) % 4, (MAIN - 1) % 2)

        @pl.when(wid < XTRA)
        def _():
            c = MAIN * NW + wid
            pltpu.sync_copy(es_hbm.at[c], es_r.at[0])
            pltpu.sync_copy(ed_hbm.at[c], ed_r.at[0])
            pltpu.async_copy(x_hbm.at[es_r.at[0]], rows[0], gsem[0]).wait()
            pltpu.sync_copy(rows[0], shared.at[ed_r.at[0]], add=True)
            count_chunk(0)

        plsc.subcore_barrier()

        base = sid * RPT
        if with_cnt:
            pltpu.async_copy(cnt_v, cnt_out.at[pl.ds(wid * N, N)], isem[0])
        pltpu.async_copy(
            shared.at[pl.ds(base, RPT)],
            sums_out.at[pl.ds(cid * N + base, RPT)],
            isem[1],
        )

        @pl.when(sid == NS - 1)
        def _():
            pltpu.sync_copy(
                shared.at[pl.ds(NS * RPT, TAIL)],
                sums_out.at[pl.ds(cid * N + NS * RPT, TAIL)],
            )

        if with_cnt:
            pltpu.make_async_copy(
                cnt_v, cnt_out.at[pl.ds(wid * N, N)], isem[0]).wait()
        pltpu.make_async_copy(
            shared.at[pl.ds(base, RPT)],
            sums_out.at[pl.ds(cid * N + base, RPT)],
            isem[1],
        ).wait()

    return pl.kernel(
        body, out_type=out_type, mesh=mesh, scratch_types=scratch,
        compiler_params=pltpu.CompilerParams(needs_layout_passes=False),
    )


def _dot(a, b):
    return lax.dot_general(
        a, b, (((1,), (0,)), ((), ())),
        precision=lax.Precision.HIGHEST,
        preferred_element_type=jnp.float32,
    )


def _mean(s0_ref, s1_ref, cnt_ref):
    s = s0_ref[...] + s1_ref[...]
    deg = jnp.sum(cnt_ref[...], axis=1)
    recip = 1.0 / jnp.maximum(deg, 1.0)
    return s * recip[:, None]


def _tc_layer1(s0_ref, s1_ref, cnt_ref, x_ref, wl_ref, wr_ref, b_ref, o_ref):
    mean = _mean(s0_ref, s1_ref, cnt_ref)
    h = _dot(mean, wl_ref[...]) + _dot(x_ref[...], wr_ref[...]) + b_ref[...]
    o_ref[...] = jnp.maximum(h, 0.0)


def _tc_layer2(s0_ref, s1_ref, cnt_ref, h_ref, wl_ref, wr_ref, b_ref,
               wo_ref, bo_ref, o_ref):
    mean = _mean(s0_ref, s1_ref, cnt_ref)
    h2 = _dot(mean, wl_ref[...]) + _dot(h_ref[...], wr_ref[...]) + b_ref[...]
    h2 = jnp.maximum(h2, 0.0)
    o_ref[...] = _dot(h2, wo_ref[...]) + bo_ref[...]


BN = 2000


def _row_block(i):
    return (i, 0)


def _tc_specs(extra_w):
    full = lambda shape: pl.BlockSpec(shape, lambda i: (0, 0))
    specs = [
        pl.BlockSpec((BN, F), _row_block),
        pl.BlockSpec((BN, F), lambda i: (i + N // BN, 0)),
        pl.BlockSpec((BN, NW), _row_block),
        pl.BlockSpec((BN, F), _row_block),
        full((F, F)), full((F, F)), full((1, F)),
    ]
    specs += extra_w
    return specs


_layer1_call = pl.pallas_call(
    _tc_layer1,
    grid=(N // BN,),
    in_specs=_tc_specs([]),
    out_specs=pl.BlockSpec((BN, F), _row_block),
    out_shape=jax.ShapeDtypeStruct((N, F), jnp.float32),
)

_layer2_call = pl.pallas_call(
    _tc_layer2,
    grid=(N // BN,),
    in_specs=_tc_specs([
        pl.BlockSpec((F, 64), lambda i: (0, 0)),
        pl.BlockSpec((1, 64), lambda i: (0, 0)),
    ]),
    out_specs=pl.BlockSpec((BN, 64), _row_block),
    out_shape=jax.ShapeDtypeStruct((N, 64), jnp.float32),
)

_sc_agg_cnt = _sc_aggregate(with_cnt=True)
_sc_agg = _sc_aggregate(with_cnt=False)


def kernel(x, edge_index, W1l, W1r, b1, W2l, W2r, b2, Wo, bo):
    es = edge_index[0].reshape(NCHUNKS, CH)
    ed = edge_index[1].reshape(NCHUNKS, CH)
    sums1, cnt = _sc_agg_cnt(x, es, ed)
    cnt = cnt.reshape(NW, N).T
    h1 = _layer1_call(sums1, sums1, cnt, x, W1l, W1r, b1.reshape(1, F))
    (sums2,) = _sc_agg(h1, es, ed)
    out = _layer2_call(sums2, sums2, cnt, h1, W2l, W2r, b2.reshape(1, F),
                       Wo, bo.reshape(1, 64))
    return out


# speedup vs baseline: 1.0469x; 1.0192x over previous
import jax
import jax.numpy as jnp
from jax import lax
from jax.experimental import pallas as pl
from jax.experimental.pallas import tpu as pltpu
from jax.experimental.pallas import tpu_sc as plsc

N = 10000
F = 128
E = 320000
CH = 128
NCHUNKS = E // CH
NC, NS, L = 2, 16, 16
NW = NC * NS
RPT = 624
TAIL = N - RPT * NS
MAIN = NCHUNKS // NW
XTRA = NCHUNKS - MAIN * NW


def _sc_aggregate(with_cnt: bool):
    mesh = plsc.VectorSubcoreMesh(
        core_axis_name="c", subcore_axis_name="s", num_cores=NC, num_subcores=NS
    )
    out_type = [jax.ShapeDtypeStruct((NC * N, F), jnp.float32)]
    scratch = [
        pltpu.VMEM((4, CH), jnp.int32),
        pltpu.VMEM((4, CH), jnp.int32),
        pltpu.VMEM((CH, F), jnp.float32),
        pltpu.VMEM((CH, F), jnp.float32),
        pltpu.VMEM_SHARED((N, F), jnp.float32),
        pltpu.SemaphoreType.DMA,
        pltpu.SemaphoreType.DMA,
        pltpu.SemaphoreType.DMA,
        pltpu.SemaphoreType.DMA,
        pltpu.SemaphoreType.DMA,
        pltpu.SemaphoreType.DMA,
        pltpu.SemaphoreType.DMA,
        pltpu.SemaphoreType.DMA,
    ]
    if with_cnt:
        out_type.append(jax.ShapeDtypeStruct((NW * N,), jnp.float32))
        scratch.append(pltpu.VMEM((N,), jnp.float32))

    def body(x_hbm, es_hbm, ed_hbm, sums_out, *rest):
        if with_cnt:
            (cnt_out, es_r, ed_r, r0, r1, shared,
             i0, i1, i2, i3, g0, g1, s0, s1, cnt_v) = rest
        else:
            (es_r, ed_r, r0, r1, shared, i0, i1, i2, i3, g0, g1, s0, s1) = rest
            cnt_out = cnt_v = None
        rows = (r0, r1)
        isem = (i0, i1, i2, i3)
        gsem = (g0, g1)
        ssem = (s0, s1)

        cid = lax.axis_index("c")
        sid = lax.axis_index("s")
        wid = sid * NC + cid

        def chunk_id(i):
            return wid + i * NW

        def fire_idx(i, q):
            c = chunk_id(i)
            pltpu.async_copy(es_hbm.at[c], es_r.at[q], isem[q])
            pltpu.async_copy(ed_hbm.at[c], ed_r.at[q], isem[q])

        fire_idx(0, 0)
        fire_idx(1, 1)

        zeros16 = jnp.zeros((L,), jnp.float32)

        def zrow(r, _):
            for j in range(F // L):
                r0[r, pl.ds(j * L, L)] = zeros16
            return 0
        lax.fori_loop(0, CH, zrow, 0)

        def zero_span(lo, ln):
            off = 0
            while ln > 0:
                step = min(ln, CH)
                pltpu.sync_copy(r0.at[pl.ds(0, step)],
                                shared.at[pl.ds(lo + off, step)])
                off += step
                ln -= step

        zero_span(sid * RPT, RPT)

        @pl.when(sid == NS - 1)
        def _():
            zero_span(NS * RPT, TAIL)

        if with_cnt:
            def zcnt(k, _):
                cnt_v[pl.ds(k * L, L)] = zeros16
                return 0
            lax.fori_loop(0, N // L, zcnt, 0)

        ones16 = jnp.ones((L,), jnp.float32)

        def wait_idx(i, q):
            c = chunk_id(i)
            pltpu.make_async_copy(es_hbm.at[c], es_r.at[q], isem[q]).wait()
            pltpu.make_async_copy(ed_hbm.at[c], ed_r.at[q], isem[q]).wait()

        def fire_gather(q, b):
            pltpu.async_copy(x_hbm.at[es_r.at[q]], rows[b], gsem[b])

        def wait_gather(q, b):
            pltpu.make_async_copy(x_hbm.at[es_r.at[q]], rows[b],
                                  gsem[b]).wait()

        def fire_scatter(q, b):
            pltpu.async_copy(rows[b], shared.at[ed_r.at[q]], ssem[b],
                             add=True)

        def wait_scatter(q, b):
            pltpu.make_async_copy(rows[b], shared.at[ed_r.at[q]],
                                  ssem[b]).wait()

        def count_chunk(q):
            if with_cnt:
                for j in range(CH // L):
                    idx16 = ed_r[q, pl.ds(j * L, L)]
                    plsc.addupdate_scatter(cnt_v, [idx16], ones16)

        def step(i, k, ws, g1, i2):
            b, bn, q = k % 2, (k + 1) % 2, k % 4
            if ws:
                wait_scatter((k - 1) % 4, bn)
            if g1:
                wait_idx(i + 1, (k + 1) % 4)
                fire_gather((k + 1) % 4, bn)
            if i2:
                fire_idx(i + 2, (k + 2) % 4)
            wait_gather(q, b)
            fire_scatter(q, b)
            count_chunk(q)

        wait_idx(0, 0)
        fire_gather(0, 0)
        plsc.subcore_barrier()

        step(0, 0, ws=False, g1=True, i2=True)
        step(1, 1, ws=True, g1=True, i2=True)
        step(2, 2, ws=True, g1=True, i2=True)
        step(3, 3, ws=True, g1=True, i2=True)

        def group(j, _):
            for k in range(4):
                step(4 + 4 * j + k, k, ws=True, g1=True, i2=True)
            return 0
        lax.fori_loop(0, (MAIN - 6) // 4, group, 0)

        step(MAIN - 2, (MAIN - 2) % 4, ws=True, g1=True, i2=False)
        step(MAIN - 1, (MAIN - 1) % 4, ws=True, g1=False, i2=False)
        wait_scatter((MAIN - 1) % 4, (MAIN - 1) % 2)

        @pl.when(wid < XTRA)
        def _():
            c = MAIN * NW + wid
            pltpu.sync_copy(es_hbm.at[c], es_r.at[0])
            pltpu.sync_copy(ed_hbm.at[c], ed_r.at[0])
            pltpu.async_copy(x_hbm.at[es_r.at[0]], rows[0], gsem[0]).wait()
            pltpu.sync_copy(rows[0], shared.at[ed_r.at[0]], add=True)
            count_chunk(0)

        plsc.subcore_barrier()

        base = sid * RPT
        if with_cnt:
            pltpu.async_copy(cnt_v, cnt_out.at[pl.ds(wid * N, N)], isem[0])
        pltpu.async_copy(
            shared.at[pl.ds(base, RPT)],
            sums_out.at[pl.ds(cid * N + base, RPT)],
            isem[1],
        )

        @pl.when(sid == NS - 1)
        def _():
            pltpu.sync_copy(
                shared.at[pl.ds(NS * RPT, TAIL)],
                sums_out.at[pl.ds(cid * N + NS * RPT, TAIL)],
            )

        if with_cnt:
            pltpu.make_async_copy(
                cnt_v, cnt_out.at[pl.ds(wid * N, N)], isem[0]).wait()
        pltpu.make_async_copy(
            shared.at[pl.ds(base, RPT)],
            sums_out.at[pl.ds(cid * N + base, RPT)],
            isem[1],
        ).wait()

    return pl.kernel(
        body, out_type=out_type, mesh=mesh, scratch_types=scratch,
        compiler_params=pltpu.CompilerParams(needs_layout_passes=False),
    )


def _dot(a, b):
    return lax.dot_general(
        a, b, (((1,), (0,)), ((), ())),
        precision=lax.Precision.HIGHEST,
        preferred_element_type=jnp.float32,
    )


def _mean(s0_ref, s1_ref, cnt_ref):
    s = s0_ref[...] + s1_ref[...]
    deg = jnp.sum(cnt_ref[...], axis=1)
    recip = 1.0 / jnp.maximum(deg, 1.0)
    return s * recip[:, None]


def _tc_lin(x_ref, w_ref, b_ref, o_ref):
    o_ref[...] = _dot(x_ref[...], w_ref[...]) + b_ref[...]


def _tc_layer1(s0_ref, s1_ref, cnt_ref, xr_ref, wl_ref, o_ref):
    mean = _mean(s0_ref, s1_ref, cnt_ref)
    o_ref[...] = jnp.maximum(_dot(mean, wl_ref[...]) + xr_ref[...], 0.0)


def _tc_layer2(s0_ref, s1_ref, cnt_ref, hr_ref, wl_ref,
               wo_ref, bo_ref, o_ref):
    mean = _mean(s0_ref, s1_ref, cnt_ref)
    h2 = jnp.maximum(_dot(mean, wl_ref[...]) + hr_ref[...], 0.0)
    o_ref[...] = _dot(h2, wo_ref[...]) + bo_ref[...]


BN = 2000


def _row_block(i):
    return (i, 0)


def _full(shape):
    return pl.BlockSpec(shape, lambda i: (0, 0))


def _tc_specs(extra_w):
    specs = [
        pl.BlockSpec((BN, F), _row_block),
        pl.BlockSpec((BN, F), lambda i: (i + N // BN, 0)),
        pl.BlockSpec((BN, NW), _row_block),
        pl.BlockSpec((BN, F), _row_block),
        _full((F, F)),
    ]
    specs += extra_w
    return specs


_lin_call = pl.pallas_call(
    _tc_lin,
    grid=(N // BN,),
    in_specs=[pl.BlockSpec((BN, F), _row_block), _full((F, F)), _full((1, F))],
    out_specs=pl.BlockSpec((BN, F), _row_block),
    out_shape=jax.ShapeDtypeStruct((N, F), jnp.float32),
)

_layer1_call = pl.pallas_call(
    _tc_layer1,
    grid=(N // BN,),
    in_specs=_tc_specs([]),
    out_specs=pl.BlockSpec((BN, F), _row_block),
    out_shape=jax.ShapeDtypeStruct((N, F), jnp.float32),
)

_layer2_call = pl.pallas_call(
    _tc_layer2,
    grid=(N // BN,),
    in_specs=_tc_specs([_full((F, 64)), _full((1, 64))]),
    out_specs=pl.BlockSpec((BN, 64), _row_block),
    out_shape=jax.ShapeDtypeStruct((N, 64), jnp.float32),
)

_sc_agg_cnt = _sc_aggregate(with_cnt=True)
_sc_agg = _sc_aggregate(with_cnt=False)


def kernel(x, edge_index, W1l, W1r, b1, W2l, W2r, b2, Wo, bo):
    es = edge_index[0].reshape(NCHUNKS, CH)
    ed = edge_index[1].reshape(NCHUNKS, CH)
    xr = _lin_call(x, W1r, b1.reshape(1, F))
    sums1, cnt = _sc_agg_cnt(x, es, ed)
    cnt = cnt.reshape(NW, N).T
    h1 = _layer1_call(sums1, sums1, cnt, xr, W1l)
    hr = _lin_call(h1, W2r, b2.reshape(1, F))
    (sums2,) = _sc_agg(h1, es, ed)
    out = _layer2_call(sums2, sums2, cnt, hr, W2l, Wo, bo.reshape(1, 64))
    return out
